# baseline (speedup 1.0000x reference)
import functools

import jax
import jax.numpy as jnp
from jax import lax
from jax.experimental import pallas as pl
from jax.experimental.pallas import tpu as pltpu
from jax.experimental.pallas import tpu_sc as plsc

CUTOFF = 5.0
N_NODES = 10000
N_EDGES = 320000
HIDDEN = 128
N_RBF = 64

NC, NS = 2, 16
NW = NC * NS
K = 64
CHUNKS = 160
SB = 8
SUPERS = CHUNKS // SB
SPAIR = SUPERS // 2
IN_PAIRS = SB // 2
E_PAD = NW * K * CHUNKS
N_PAD = 10240
ROWS_PER_TILE = N_PAD // NS


def _filter_body(d_ref, c_ref, g_ref, w1_ref, b1_ref, w2_ref, b2_ref, o_ref):
    d = d_ref[...]
    g = g_ref[0, 0]
    diff = d - c_ref[...]
    rbf = jnp.exp(-g * diff * diff)
    h = jnp.dot(rbf, w1_ref[...], preferred_element_type=jnp.float32) + b1_ref[...]
    h = h * jax.nn.sigmoid(h)
    w = jnp.dot(h, w2_ref[...], preferred_element_type=jnp.float32) + b2_ref[...]
    xc = jnp.clip(d * (1.0 / CUTOFF), 0.0, 1.0)
    cc = 0.5 * (jnp.cos(jnp.pi * xc) + 1.0) * (xc < 1.0).astype(jnp.float32)
    o_ref[...] = w * cc


def _filter_call(dist_pad, centers, gamma, W1, b1, W2, b2):
    BE = 2048
    return pl.pallas_call(
        _filter_body,
        grid=(E_PAD // BE,),
        in_specs=[
            pl.BlockSpec((BE, 1), lambda i: (i, 0)),
            pl.BlockSpec((1, N_RBF), lambda i: (0, 0)),
            pl.BlockSpec(memory_space=pltpu.SMEM),
            pl.BlockSpec((N_RBF, HIDDEN), lambda i: (0, 0)),
            pl.BlockSpec((1, HIDDEN), lambda i: (0, 0)),
            pl.BlockSpec((HIDDEN, HIDDEN), lambda i: (0, 0)),
            pl.BlockSpec((1, HIDDEN), lambda i: (0, 0)),
        ],
        out_specs=pl.BlockSpec((BE, HIDDEN), lambda i: (i, 0)),
        out_shape=jax.ShapeDtypeStruct((E_PAD, HIDDEN), jnp.float32),
    )(
        dist_pad.reshape(E_PAD, 1),
        centers.reshape(1, N_RBF),
        gamma.reshape(1, 1),
        W1,
        b1.reshape(1, HIDDEN),
        W2,
        b2.reshape(1, HIDDEN),
    )


def _xd_body(x_ref, wd_ref, o_ref):
    o_ref[...] = jnp.dot(x_ref[...], wd_ref[...],
                         preferred_element_type=jnp.float32)


def _xd_call(x, Wd):
    BN = 2000
    return pl.pallas_call(
        _xd_body,
        grid=(N_NODES // BN,),
        in_specs=[
            pl.BlockSpec((BN, HIDDEN), lambda i: (i, 0)),
            pl.BlockSpec((HIDDEN, HIDDEN), lambda i: (0, 0)),
        ],
        out_specs=pl.BlockSpec((BN, HIDDEN), lambda i: (i, 0)),
        out_shape=jax.ShapeDtypeStruct((N_NODES, HIDDEN), jnp.float32),
    )(x, Wd)


def _combine_body(a_ref, b_ref, o_ref):
    o_ref[...] = a_ref[...] + b_ref[...]


def _combine_call(p0, p1):
    BN = 2000
    return pl.pallas_call(
        _combine_body,
        grid=(N_NODES // BN,),
        in_specs=[
            pl.BlockSpec((BN, HIDDEN), lambda i: (i, 0)),
            pl.BlockSpec((BN, HIDDEN), lambda i: (i, 0)),
        ],
        out_specs=pl.BlockSpec((BN, HIDDEN), lambda i: (i, 0)),
        out_shape=jax.ShapeDtypeStruct((N_NODES, HIDDEN), jnp.float32),
    )(p0, p1)


def _sc_body(xd_h, wc_h, src2_h, dst2_h, out_h,
             rowsA, wcbA, rowsB, wcbB,
             src_s0, dst_s0,
             src_s1, dst_s1,
             acc, gsemA, gsemB, ssemA, ssemB, psem0, psem1):
    c = lax.axis_index("c")
    s = lax.axis_index("s")
    wid = c * NS + s
    row_w = wid * CHUNKS

    @plsc.parallel_loop(0, K)
    def _zrow(i):
        for j in range(HIDDEN // 16):
            rowsA[i, pl.ds(j * 16, 16)] = jnp.zeros((16,), jnp.float32)

    zbase = s * ROWS_PER_TILE
    n_full = ROWS_PER_TILE // K
    z_rem = ROWS_PER_TILE - n_full * K
    for t in range(n_full):
        pltpu.sync_copy(rowsA, acc.at[pl.ds(zbase + t * K, K)])
    if z_rem:
        pltpu.sync_copy(rowsA.at[pl.ds(0, z_rem)],
                        acc.at[pl.ds(zbase + n_full * K, z_rem)])

    sb0 = (src_s0, dst_s0)
    sb1 = (src_s1, dst_s1)
    cbA = (rowsA, wcbA, gsemA, ssemA)
    cbB = (rowsB, wcbB, gsemB, ssemB)

    def _preload(sup, sbuf, psem):
        r0 = row_w + sup * SB
        pltpu.async_copy(src2_h.at[pl.ds(r0, SB)], sbuf[0], psem)
        pltpu.async_copy(dst2_h.at[pl.ds(r0, SB)], sbuf[1], psem)

    def _pwait(sup, sbuf, psem):
        r0 = row_w + sup * SB
        pltpu.make_async_copy(src2_h.at[pl.ds(r0, SB)], sbuf[0], psem).wait()
        pltpu.make_async_copy(dst2_h.at[pl.ds(r0, SB)], sbuf[1], psem).wait()

    def _sc_wait(ci, sbuf, cbuf):
        rows, _, _, ssem = cbuf
        pltpu.make_async_copy(rows, acc.at[sbuf[1].at[ci]], ssem).wait()

    def _startc(sup, ci, sbuf, cbuf):
        rows, wcb, gsem, _ = cbuf
        base = (row_w + sup * SB + ci) * K
        pltpu.async_copy(xd_h.at[sbuf[0].at[ci]], rows, gsem)
        pltpu.async_copy(wc_h.at[pl.ds(base, K)], wcb, gsem)

    def _finishc(sup, ci, sbuf, cbuf):
        rows, wcb, gsem, ssem = cbuf
        base = (row_w + sup * SB + ci) * K
        pltpu.make_async_copy(xd_h.at[sbuf[0].at[ci]], rows, gsem).wait()
        pltpu.make_async_copy(wc_h.at[pl.ds(base, K)], wcb, gsem).wait()

        @plsc.parallel_loop(0, K, unroll=2)
        def _mul(i):
            for j in range(HIDDEN // 16):
                sl = pl.ds(j * 16, 16)
                rows[i, sl] = rows[i, sl] * wcb[i, sl]

        pltpu.async_copy(rows, acc.at[sbuf[1].at[ci]], ssem, add=True)

    def _super(sup, sbuf, psem):
        _pwait(sup, sbuf, psem)
        _startc(sup, 0, sbuf, cbA)

        def _ip(p, carry):
            @pl.when(p > 0)
            def _():
                _sc_wait(2 * p - 1, sbuf, cbB)
            _startc(sup, 2 * p + 1, sbuf, cbB)
            _finishc(sup, 2 * p, sbuf, cbA)

            @pl.when(p < IN_PAIRS - 1)
            def _():
                _sc_wait(2 * p, sbuf, cbA)
                _startc(sup, 2 * p + 2, sbuf, cbA)
            _finishc(sup, 2 * p + 1, sbuf, cbB)
            return carry
        lax.fori_loop(0, IN_PAIRS, _ip, 0)
        _sc_wait(SB - 2, sbuf, cbA)
        _sc_wait(SB - 1, sbuf, cbB)

    _preload(0, sb0, psem0)
    plsc.subcore_barrier()

    def _op(t, carry):
        _preload(2 * t + 1, sb1, psem1)
        _super(2 * t, sb0, psem0)

        @pl.when(t < SPAIR - 1)
        def _():
            _preload(2 * t + 2, sb0, psem0)
        _super(2 * t + 1, sb1, psem1)
        return carry
    lax.fori_loop(0, SPAIR, _op, 0)
    plsc.subcore_barrier()

    for t in range(n_full):
        pltpu.sync_copy(acc.at[pl.ds(zbase + t * K, K)], rowsA)
        pltpu.sync_copy(rowsA, out_h.at[c, pl.ds(zbase + t * K, K)])
    if z_rem:
        pltpu.sync_copy(acc.at[pl.ds(zbase + n_full * K, z_rem)],
                        rowsA.at[pl.ds(0, z_rem)])
        pltpu.sync_copy(rowsA.at[pl.ds(0, z_rem)],
                        out_h.at[c, pl.ds(zbase + n_full * K, z_rem)])


def _sc_scratch():
    sbuf = [
        pltpu.VMEM((SB, K), jnp.int32),
        pltpu.VMEM((SB, K), jnp.int32),
    ]
    return ([pltpu.VMEM((K, HIDDEN), jnp.float32),
             pltpu.VMEM((K, HIDDEN), jnp.float32),
             pltpu.VMEM((K, HIDDEN), jnp.float32),
             pltpu.VMEM((K, HIDDEN), jnp.float32)]
            + sbuf + sbuf
            + [pltpu.VMEM_SHARED((N_PAD, HIDDEN), jnp.float32)]
            + [pltpu.SemaphoreType.DMA] * 6)


_sc_call = functools.partial(
    pl.kernel,
    out_type=jax.ShapeDtypeStruct((NC, N_PAD, HIDDEN), jnp.float32),
    mesh=plsc.VectorSubcoreMesh(core_axis_name="c", subcore_axis_name="s"),
    scratch_types=_sc_scratch(),
)(_sc_body)


def kernel(x, edge_index, distances, centers, gamma, W1, b1, W2, b2, Wd):
    src = edge_index[0].astype(jnp.int32)
    dst = edge_index[1].astype(jnp.int32)
    pad = E_PAD - N_EDGES
    dist_pad = jnp.concatenate(
        [distances, jnp.full((pad,), CUTOFF, jnp.float32)])
    src_p = jnp.concatenate([src, jnp.zeros((pad,), jnp.int32)])
    dst_p = jnp.concatenate([dst, jnp.zeros((pad,), jnp.int32)])

    wc = _filter_call(dist_pad, centers.astype(jnp.float32),
                      gamma.astype(jnp.float32), W1, b1, W2, b2)
    xd = _xd_call(x, Wd)
    parts = _sc_call(xd, wc,
                     src_p.reshape(-1, K),
                     dst_p.reshape(-1, K))
    return _combine_call(parts[0], parts[1])


# speedup vs baseline: 1.2149x; 1.2149x over previous
import functools

import jax
import jax.numpy as jnp
from jax import lax
from jax.experimental import pallas as pl
from jax.experimental.pallas import tpu as pltpu
from jax.experimental.pallas import tpu_sc as plsc

CUTOFF = 5.0
N_NODES = 10000
N_EDGES = 320000
HIDDEN = 128
N_RBF = 64

NC, NS = 2, 16
NW = NC * NS
K = 80
CHUNKS = 126
E_PAD = NW * K * CHUNKS
N_PAD = 10240
ROWS_PER_TILE = N_PAD // NS


def _filter_body(d_ref, c_ref, g_ref, w1_ref, b1_ref, w2_ref, b2_ref, o_ref):
    d = d_ref[...]
    g = g_ref[0, 0]
    diff = d - c_ref[...]
    rbf = jnp.exp(-g * diff * diff)
    h = jnp.dot(rbf, w1_ref[...], preferred_element_type=jnp.float32) + b1_ref[...]
    h = h * jax.nn.sigmoid(h)
    w = jnp.dot(h, w2_ref[...], preferred_element_type=jnp.float32) + b2_ref[...]
    xc = jnp.clip(d * (1.0 / CUTOFF), 0.0, 1.0)
    cc = 0.5 * (jnp.cos(jnp.pi * xc) + 1.0) * (xc < 1.0).astype(jnp.float32)
    o_ref[...] = w * cc


def _filter_call(dist_pad, centers, gamma, W1, b1, W2, b2):
    BE = 2560
    return pl.pallas_call(
        _filter_body,
        grid=(E_PAD // BE,),
        in_specs=[
            pl.BlockSpec((BE, 1), lambda i: (i, 0)),
            pl.BlockSpec((1, N_RBF), lambda i: (0, 0)),
            pl.BlockSpec(memory_space=pltpu.SMEM),
            pl.BlockSpec((N_RBF, HIDDEN), lambda i: (0, 0)),
            pl.BlockSpec((1, HIDDEN), lambda i: (0, 0)),
            pl.BlockSpec((HIDDEN, HIDDEN), lambda i: (0, 0)),
            pl.BlockSpec((1, HIDDEN), lambda i: (0, 0)),
        ],
        out_specs=pl.BlockSpec((BE, HIDDEN), lambda i: (i, 0)),
        out_shape=jax.ShapeDtypeStruct((E_PAD, HIDDEN), jnp.float32),
    )(
        dist_pad.reshape(E_PAD, 1),
        centers.reshape(1, N_RBF),
        gamma.reshape(1, 1),
        W1,
        b1.reshape(1, HIDDEN),
        W2,
        b2.reshape(1, HIDDEN),
    )


def _xd_body(x_ref, wd_ref, o_ref):
    o_ref[...] = jnp.dot(x_ref[...], wd_ref[...],
                         preferred_element_type=jnp.float32)


def _xd_call(x, Wd):
    BN = 2000
    return pl.pallas_call(
        _xd_body,
        grid=(N_NODES // BN,),
        in_specs=[
            pl.BlockSpec((BN, HIDDEN), lambda i: (i, 0)),
            pl.BlockSpec((HIDDEN, HIDDEN), lambda i: (0, 0)),
        ],
        out_specs=pl.BlockSpec((BN, HIDDEN), lambda i: (i, 0)),
        out_shape=jax.ShapeDtypeStruct((N_NODES, HIDDEN), jnp.float32),
    )(x, Wd)


def _combine_body(a_ref, b_ref, o_ref):
    o_ref[...] = a_ref[...] + b_ref[...]


def _combine_call(p0, p1):
    BN = 2000
    return pl.pallas_call(
        _combine_body,
        grid=(N_NODES // BN,),
        in_specs=[
            pl.BlockSpec((BN, HIDDEN), lambda i: (i, 0)),
            pl.BlockSpec((BN, HIDDEN), lambda i: (i, 0)),
        ],
        out_specs=pl.BlockSpec((BN, HIDDEN), lambda i: (i, 0)),
        out_shape=jax.ShapeDtypeStruct((N_NODES, HIDDEN), jnp.float32),
    )(p0, p1)


def _sc_body(xd_h, wc_h, src_h, dst_h, out_h,
             idx_s0, idx_d0, rows0, wcb0,
             idx_s1, idx_d1, rows1, wcb1,
             acc, sem0, sem1):
    c = lax.axis_index("c")
    s = lax.axis_index("s")
    wid = c * NS + s
    base_w = wid * CHUNKS * K

    @plsc.parallel_loop(0, K)
    def _zrow(i):
        for j in range(HIDDEN // 16):
            rows0[i, pl.ds(j * 16, 16)] = jnp.zeros((16,), jnp.float32)

    zbase = s * ROWS_PER_TILE
    n_full = ROWS_PER_TILE // K
    for t in range(n_full):
        pltpu.sync_copy(rows0, acc.at[pl.ds(zbase + t * K, K)])
    plsc.subcore_barrier()

    def _start(ci, idx_s, idx_d, rows, wcb, sem):
        base = base_w + ci * K
        pltpu.sync_copy(src_h.at[pl.ds(base, K)], idx_s)
        pltpu.sync_copy(dst_h.at[pl.ds(base, K)], idx_d)
        pltpu.async_copy(xd_h.at[idx_s], rows, sem)
        pltpu.async_copy(wc_h.at[pl.ds(base, K)], wcb, sem)

    def _finish(idx_s, idx_d, rows, wcb, sem):
        pltpu.make_async_copy(xd_h.at[idx_s], rows, sem).wait()
        pltpu.make_async_copy(wc_h.at[pl.ds(0, K)], wcb, sem).wait()

        @plsc.parallel_loop(0, K, unroll=2)
        def _mulrow(i):
            for j in range(HIDDEN // 16):
                sl = pl.ds(j * 16, 16)
                rows[i, sl] = rows[i, sl] * wcb[i, sl]

        pltpu.sync_copy(rows, acc.at[idx_d], add=True)

    buf0 = (idx_s0, idx_d0, rows0, wcb0, sem0)
    buf1 = (idx_s1, idx_d1, rows1, wcb1, sem1)
    _start(0, *buf0)

    def _pair(j, carry):
        _start(2 * j + 1, *buf1)
        _finish(*buf0)

        @pl.when(j < CHUNKS // 2 - 1)
        def _():
            _start(2 * j + 2, *buf0)
        _finish(*buf1)
        return carry
    lax.fori_loop(0, CHUNKS // 2, _pair, 0)
    plsc.subcore_barrier()

    for t in range(n_full):
        pltpu.sync_copy(acc.at[pl.ds(zbase + t * K, K)], rows0)
        pltpu.sync_copy(rows0, out_h.at[c, pl.ds(zbase + t * K, K)])


_sc_call = functools.partial(
    pl.kernel,
    out_type=jax.ShapeDtypeStruct((NC, N_PAD, HIDDEN), jnp.float32),
    mesh=plsc.VectorSubcoreMesh(core_axis_name="c", subcore_axis_name="s"),
    scratch_types=[
        pltpu.VMEM((K,), jnp.int32),
        pltpu.VMEM((K,), jnp.int32),
        pltpu.VMEM((K, HIDDEN), jnp.float32),
        pltpu.VMEM((K, HIDDEN), jnp.float32),
        pltpu.VMEM((K,), jnp.int32),
        pltpu.VMEM((K,), jnp.int32),
        pltpu.VMEM((K, HIDDEN), jnp.float32),
        pltpu.VMEM((K, HIDDEN), jnp.float32),
        pltpu.VMEM_SHARED((N_PAD, HIDDEN), jnp.float32),
        pltpu.SemaphoreType.DMA,
        pltpu.SemaphoreType.DMA,
    ],
)(_sc_body)


def kernel(x, edge_index, distances, centers, gamma, W1, b1, W2, b2, Wd):
    src = edge_index[0].astype(jnp.int32)
    dst = edge_index[1].astype(jnp.int32)
    pad = E_PAD - N_EDGES
    dist_pad = jnp.concatenate(
        [distances, jnp.full((pad,), CUTOFF, jnp.float32)])
    src_p = jnp.concatenate([src, jnp.zeros((pad,), jnp.int32)])
    dst_p = jnp.concatenate([dst, jnp.zeros((pad,), jnp.int32)])

    wc = _filter_call(dist_pad, centers.astype(jnp.float32),
                      gamma.astype(jnp.float32), W1, b1, W2, b2)
    xd = _xd_call(x, Wd)
    parts = _sc_call(xd, wc, src_p, dst_p)
    return _combine_call(parts[0], parts[1])


# speedup vs baseline: 1.2530x; 1.0313x over previous
import functools

import jax
import jax.numpy as jnp
from jax import lax
from jax.experimental import pallas as pl
from jax.experimental.pallas import tpu as pltpu
from jax.experimental.pallas import tpu_sc as plsc

CUTOFF = 5.0
N_NODES = 10000
N_EDGES = 320000
HIDDEN = 128
N_RBF = 64

NC, NS = 2, 16
NW = NC * NS
K = 80
CHUNKS = 64
PHASES = 2
E_HALF = NW * K * CHUNKS
E_PAD = PHASES * E_HALF
N_PAD = 10240
ROWS_PER_TILE = N_PAD // NS


def _filter_body(d_ref, c_ref, g_ref, w1_ref, b1_ref, w2_ref, b2_ref, o_ref):
    d = d_ref[...]
    g = g_ref[0, 0]
    diff = d - c_ref[...]
    rbf = jnp.exp(-g * diff * diff)
    h = jnp.dot(rbf, w1_ref[...], preferred_element_type=jnp.float32) + b1_ref[...]
    h = h * jax.nn.sigmoid(h)
    w = jnp.dot(h, w2_ref[...], preferred_element_type=jnp.float32) + b2_ref[...]
    xc = jnp.clip(d * (1.0 / CUTOFF), 0.0, 1.0)
    cc = 0.5 * (jnp.cos(jnp.pi * xc) + 1.0) * (xc < 1.0).astype(jnp.float32)
    o_ref[...] = w * cc


def _filter_call(dist_half, centers, gamma, W1, b1, W2, b2):
    BE = 2048
    return pl.pallas_call(
        _filter_body,
        grid=(E_HALF // BE,),
        in_specs=[
            pl.BlockSpec((BE, 1), lambda i: (i, 0)),
            pl.BlockSpec((1, N_RBF), lambda i: (0, 0)),
            pl.BlockSpec(memory_space=pltpu.SMEM),
            pl.BlockSpec((N_RBF, HIDDEN), lambda i: (0, 0)),
            pl.BlockSpec((1, HIDDEN), lambda i: (0, 0)),
            pl.BlockSpec((HIDDEN, HIDDEN), lambda i: (0, 0)),
            pl.BlockSpec((1, HIDDEN), lambda i: (0, 0)),
        ],
        out_specs=pl.BlockSpec((BE, HIDDEN), lambda i: (i, 0)),
        out_shape=jax.ShapeDtypeStruct((E_HALF, HIDDEN), jnp.float32),
    )(
        dist_half.reshape(E_HALF, 1),
        centers.reshape(1, N_RBF),
        gamma.reshape(1, 1),
        W1,
        b1.reshape(1, HIDDEN),
        W2,
        b2.reshape(1, HIDDEN),
    )


def _xd_body(x_ref, wd_ref, o_ref):
    o_ref[...] = jnp.dot(x_ref[...], wd_ref[...],
                         preferred_element_type=jnp.float32)


def _xd_call(x, Wd):
    BN = 2000
    return pl.pallas_call(
        _xd_body,
        grid=(N_NODES // BN,),
        in_specs=[
            pl.BlockSpec((BN, HIDDEN), lambda i: (i, 0)),
            pl.BlockSpec((HIDDEN, HIDDEN), lambda i: (0, 0)),
        ],
        out_specs=pl.BlockSpec((BN, HIDDEN), lambda i: (i, 0)),
        out_shape=jax.ShapeDtypeStruct((N_NODES, HIDDEN), jnp.float32),
    )(x, Wd)


def _combine_body(a_ref, b_ref, c_ref, d_ref, o_ref):
    o_ref[...] = (a_ref[...] + b_ref[...]) + (c_ref[...] + d_ref[...])


def _combine_call(p0, p1, p2, p3):
    BN = 2000
    spec = pl.BlockSpec((BN, HIDDEN), lambda i: (i, 0))
    return pl.pallas_call(
        _combine_body,
        grid=(N_NODES // BN,),
        in_specs=[spec, spec, spec, spec],
        out_specs=spec,
        out_shape=jax.ShapeDtypeStruct((N_NODES, HIDDEN), jnp.float32),
    )(p0, p1, p2, p3)


def _sc_body(xd_h, wc_h, src_h, dst_h, out_h,
             idx_s0, idx_d0, rows0, wcb0,
             idx_s1, idx_d1, rows1, wcb1,
             acc, sem0, sem1):
    c = lax.axis_index("c")
    s = lax.axis_index("s")
    wid = c * NS + s
    base_w = wid * CHUNKS * K

    @plsc.parallel_loop(0, K)
    def _zrow(i):
        for j in range(HIDDEN // 16):
            rows0[i, pl.ds(j * 16, 16)] = jnp.zeros((16,), jnp.float32)

    zbase = s * ROWS_PER_TILE
    n_full = ROWS_PER_TILE // K
    for t in range(n_full):
        pltpu.sync_copy(rows0, acc.at[pl.ds(zbase + t * K, K)])
    plsc.subcore_barrier()

    def _start(ci, idx_s, idx_d, rows, wcb, sem):
        base = base_w + ci * K
        pltpu.sync_copy(src_h.at[pl.ds(base, K)], idx_s)
        pltpu.sync_copy(dst_h.at[pl.ds(base, K)], idx_d)
        pltpu.async_copy(xd_h.at[idx_s], rows, sem)
        pltpu.async_copy(wc_h.at[pl.ds(base, K)], wcb, sem)

    def _finish(idx_s, idx_d, rows, wcb, sem):
        pltpu.make_async_copy(xd_h.at[idx_s], rows, sem).wait()
        pltpu.make_async_copy(wc_h.at[pl.ds(0, K)], wcb, sem).wait()

        @plsc.parallel_loop(0, K, unroll=2)
        def _mulrow(i):
            for j in range(HIDDEN // 16):
                sl = pl.ds(j * 16, 16)
                rows[i, sl] = rows[i, sl] * wcb[i, sl]

        pltpu.sync_copy(rows, acc.at[idx_d], add=True)

    buf0 = (idx_s0, idx_d0, rows0, wcb0, sem0)
    buf1 = (idx_s1, idx_d1, rows1, wcb1, sem1)
    _start(0, *buf0)

    def _pair(j, carry):
        _start(2 * j + 1, *buf1)
        _finish(*buf0)

        @pl.when(j < CHUNKS // 2 - 1)
        def _():
            _start(2 * j + 2, *buf0)
        _finish(*buf1)
        return carry
    lax.fori_loop(0, CHUNKS // 2, _pair, 0)
    plsc.subcore_barrier()

    for t in range(n_full):
        pltpu.sync_copy(acc.at[pl.ds(zbase + t * K, K)], rows0)
        pltpu.sync_copy(rows0, out_h.at[c, pl.ds(zbase + t * K, K)])


_sc_call = functools.partial(
    pl.kernel,
    out_type=jax.ShapeDtypeStruct((NC, N_PAD, HIDDEN), jnp.float32),
    mesh=plsc.VectorSubcoreMesh(core_axis_name="c", subcore_axis_name="s"),
    scratch_types=[
        pltpu.VMEM((K,), jnp.int32),
        pltpu.VMEM((K,), jnp.int32),
        pltpu.VMEM((K, HIDDEN), jnp.float32),
        pltpu.VMEM((K, HIDDEN), jnp.float32),
        pltpu.VMEM((K,), jnp.int32),
        pltpu.VMEM((K,), jnp.int32),
        pltpu.VMEM((K, HIDDEN), jnp.float32),
        pltpu.VMEM((K, HIDDEN), jnp.float32),
        pltpu.VMEM_SHARED((N_PAD, HIDDEN), jnp.float32),
        pltpu.SemaphoreType.DMA,
        pltpu.SemaphoreType.DMA,
    ],
)(_sc_body)


def kernel(x, edge_index, distances, centers, gamma, W1, b1, W2, b2, Wd):
    src = edge_index[0].astype(jnp.int32)
    dst = edge_index[1].astype(jnp.int32)
    pad = E_PAD - N_EDGES
    dist_pad = jnp.concatenate(
        [distances, jnp.full((pad,), CUTOFF, jnp.float32)])
    src_p = jnp.concatenate([src, jnp.zeros((pad,), jnp.int32)])
    dst_p = jnp.concatenate([dst, jnp.zeros((pad,), jnp.int32)])

    xd = _xd_call(x, Wd)
    centers32 = centers.astype(jnp.float32)
    gamma32 = gamma.astype(jnp.float32)
    partials = []
    for ph in range(PHASES):
        sl = slice(ph * E_HALF, (ph + 1) * E_HALF)
        wc = _filter_call(dist_pad[sl], centers32, gamma32, W1, b1, W2, b2)
        parts = _sc_call(xd, wc, src_p[sl], dst_p[sl])
        partials.extend([parts[0], parts[1]])
    return _combine_call(*partials)


# speedup vs baseline: 1.4150x; 1.1293x over previous
import functools

import jax
import jax.numpy as jnp
from jax import lax
from jax.experimental import pallas as pl
from jax.experimental.pallas import tpu as pltpu
from jax.experimental.pallas import tpu_sc as plsc

CUTOFF = 5.0
N_NODES = 10000
N_EDGES = 320000
HIDDEN = 128
N_RBF = 64

NC, NS = 2, 16
NW = NC * NS
K = 80
CHUNKS = 32
PHASES = 4
E_HALF = NW * K * CHUNKS
E_PAD = PHASES * E_HALF
N_PAD = 10240
ROWS_PER_TILE = N_PAD // NS


def _filter_body(d_ref, c_ref, g_ref, w1_ref, b1_ref, w2_ref, b2_ref, o_ref):
    d = d_ref[...]
    g = g_ref[0, 0]
    diff = d - c_ref[...]
    rbf = jnp.exp(-g * diff * diff)
    h = jnp.dot(rbf, w1_ref[...], preferred_element_type=jnp.float32) + b1_ref[...]
    h = h * jax.nn.sigmoid(h)
    w = jnp.dot(h, w2_ref[...], preferred_element_type=jnp.float32) + b2_ref[...]
    xc = jnp.clip(d * (1.0 / CUTOFF), 0.0, 1.0)
    cc = 0.5 * (jnp.cos(jnp.pi * xc) + 1.0) * (xc < 1.0).astype(jnp.float32)
    o_ref[...] = w * cc


def _filter_call(dist_half, centers, gamma, W1, b1, W2, b2):
    BE = 2048
    return pl.pallas_call(
        _filter_body,
        grid=(E_HALF // BE,),
        in_specs=[
            pl.BlockSpec((BE, 1), lambda i: (i, 0)),
            pl.BlockSpec((1, N_RBF), lambda i: (0, 0)),
            pl.BlockSpec(memory_space=pltpu.SMEM),
            pl.BlockSpec((N_RBF, HIDDEN), lambda i: (0, 0)),
            pl.BlockSpec((1, HIDDEN), lambda i: (0, 0)),
            pl.BlockSpec((HIDDEN, HIDDEN), lambda i: (0, 0)),
            pl.BlockSpec((1, HIDDEN), lambda i: (0, 0)),
        ],
        out_specs=pl.BlockSpec((BE, HIDDEN), lambda i: (i, 0)),
        out_shape=jax.ShapeDtypeStruct((E_HALF, HIDDEN), jnp.float32),
    )(
        dist_half.reshape(E_HALF, 1),
        centers.reshape(1, N_RBF),
        gamma.reshape(1, 1),
        W1,
        b1.reshape(1, HIDDEN),
        W2,
        b2.reshape(1, HIDDEN),
    )


def _xd_body(x_ref, wd_ref, o_ref):
    o_ref[...] = jnp.dot(x_ref[...], wd_ref[...],
                         preferred_element_type=jnp.float32)


def _xd_call(x, Wd):
    BN = 2000
    return pl.pallas_call(
        _xd_body,
        grid=(N_NODES // BN,),
        in_specs=[
            pl.BlockSpec((BN, HIDDEN), lambda i: (i, 0)),
            pl.BlockSpec((HIDDEN, HIDDEN), lambda i: (0, 0)),
        ],
        out_specs=pl.BlockSpec((BN, HIDDEN), lambda i: (i, 0)),
        out_shape=jax.ShapeDtypeStruct((N_NODES, HIDDEN), jnp.float32),
    )(x, Wd)


def _combine_body(*refs):
    o_ref = refs[-1]
    total = refs[0][...]
    for r in refs[1:-1]:
        total = total + r[...]
    o_ref[...] = total


def _combine_call(*parts):
    BN = 2000
    spec = pl.BlockSpec((BN, HIDDEN), lambda i: (i, 0))
    return pl.pallas_call(
        _combine_body,
        grid=(N_NODES // BN,),
        in_specs=[spec] * len(parts),
        out_specs=spec,
        out_shape=jax.ShapeDtypeStruct((N_NODES, HIDDEN), jnp.float32),
    )(*parts)


def _sc_body(xd_h, wc_h, src_h, dst_h, out_h,
             idx_s0, idx_d0, rows0, wcb0,
             idx_s1, idx_d1, rows1, wcb1,
             acc, sem0, sem1):
    c = lax.axis_index("c")
    s = lax.axis_index("s")
    wid = c * NS + s
    base_w = wid * CHUNKS * K

    @plsc.parallel_loop(0, K)
    def _zrow(i):
        for j in range(HIDDEN // 16):
            rows0[i, pl.ds(j * 16, 16)] = jnp.zeros((16,), jnp.float32)

    zbase = s * ROWS_PER_TILE
    n_full = ROWS_PER_TILE // K
    for t in range(n_full):
        pltpu.sync_copy(rows0, acc.at[pl.ds(zbase + t * K, K)])
    plsc.subcore_barrier()

    def _start(ci, idx_s, idx_d, rows, wcb, sem):
        base = base_w + ci * K
        pltpu.sync_copy(src_h.at[pl.ds(base, K)], idx_s)
        pltpu.sync_copy(dst_h.at[pl.ds(base, K)], idx_d)
        pltpu.async_copy(xd_h.at[idx_s], rows, sem)
        pltpu.async_copy(wc_h.at[pl.ds(base, K)], wcb, sem)

    def _finish(idx_s, idx_d, rows, wcb, sem):
        pltpu.make_async_copy(xd_h.at[idx_s], rows, sem).wait()
        pltpu.make_async_copy(wc_h.at[pl.ds(0, K)], wcb, sem).wait()

        @plsc.parallel_loop(0, K, unroll=2)
        def _mulrow(i):
            for j in range(HIDDEN // 16):
                sl = pl.ds(j * 16, 16)
                rows[i, sl] = rows[i, sl] * wcb[i, sl]

        pltpu.sync_copy(rows, acc.at[idx_d], add=True)

    buf0 = (idx_s0, idx_d0, rows0, wcb0, sem0)
    buf1 = (idx_s1, idx_d1, rows1, wcb1, sem1)
    _start(0, *buf0)

    def _pair(j, carry):
        _start(2 * j + 1, *buf1)
        _finish(*buf0)

        @pl.when(j < CHUNKS // 2 - 1)
        def _():
            _start(2 * j + 2, *buf0)
        _finish(*buf1)
        return carry
    lax.fori_loop(0, CHUNKS // 2, _pair, 0)
    plsc.subcore_barrier()

    for t in range(n_full):
        pltpu.sync_copy(acc.at[pl.ds(zbase + t * K, K)], rows0)
        pltpu.sync_copy(rows0, out_h.at[c, pl.ds(zbase + t * K, K)])


_sc_call = functools.partial(
    pl.kernel,
    out_type=jax.ShapeDtypeStruct((NC, N_PAD, HIDDEN), jnp.float32),
    mesh=plsc.VectorSubcoreMesh(core_axis_name="c", subcore_axis_name="s"),
    scratch_types=[
        pltpu.VMEM((K,), jnp.int32),
        pltpu.VMEM((K,), jnp.int32),
        pltpu.VMEM((K, HIDDEN), jnp.float32),
        pltpu.VMEM((K, HIDDEN), jnp.float32),
        pltpu.VMEM((K,), jnp.int32),
        pltpu.VMEM((K,), jnp.int32),
        pltpu.VMEM((K, HIDDEN), jnp.float32),
        pltpu.VMEM((K, HIDDEN), jnp.float32),
        pltpu.VMEM_SHARED((N_PAD, HIDDEN), jnp.float32),
        pltpu.SemaphoreType.DMA,
        pltpu.SemaphoreType.DMA,
    ],
)(_sc_body)


def kernel(x, edge_index, distances, centers, gamma, W1, b1, W2, b2, Wd):
    src = edge_index[0].astype(jnp.int32)
    dst = edge_index[1].astype(jnp.int32)
    pad = E_PAD - N_EDGES
    dist_pad = jnp.concatenate(
        [distances, jnp.full((pad,), CUTOFF, jnp.float32)])
    src_p = jnp.concatenate([src, jnp.zeros((pad,), jnp.int32)])
    dst_p = jnp.concatenate([dst, jnp.zeros((pad,), jnp.int32)])

    xd = _xd_call(x, Wd)
    centers32 = centers.astype(jnp.float32)
    gamma32 = gamma.astype(jnp.float32)
    partials = []
    for ph in range(PHASES):
        sl = slice(ph * E_HALF, (ph + 1) * E_HALF)
        wc = _filter_call(dist_pad[sl], centers32, gamma32, W1, b1, W2, b2)
        parts = _sc_call(xd, wc, src_p[sl], dst_p[sl])
        partials.extend([parts[0], parts[1]])
    return _combine_call(*partials)
